# baseline (speedup 1.0000x reference)
import functools

import jax
import jax.numpy as jnp
from jax import lax
from jax.experimental import pallas as pl
from jax.experimental.pallas import tpu as pltpu
from jax.experimental.pallas import tpu_sc as plsc

B = 16384
V = 1000000
D = 64
P = 64

_NC = 2
_NS = 16
_NW = _NC * _NS
_B_PER_W = B // _NW


@functools.cache
def _make_sc_gather():
    mesh = plsc.VectorSubcoreMesh(core_axis_name="c", subcore_axis_name="s")

    @functools.partial(
        pl.kernel,
        mesh=mesh,
        out_type=jax.ShapeDtypeStruct((B * D,), jnp.float32),
        scratch_types=[
            pltpu.VMEM((_B_PER_W,), jnp.int32),
            pltpu.VMEM((D, _B_PER_W), jnp.int32),
            pltpu.VMEM((D * _B_PER_W,), jnp.float32),
            pltpu.SemaphoreType.DMA,
        ],
        compiler_params=pltpu.CompilerParams(use_tc_tiling_on_sc=False),
    )
    def gather_kernel(tblT_hbm, ids_hbm, out_hbm, ids_v, widx_v, stage_v,
                      sem):
        wid = lax.axis_index("s") * _NC + lax.axis_index("c")
        base = wid * _B_PER_W
        pltpu.sync_copy(ids_hbm.at[pl.ds(base, _B_PER_W)], ids_v)
        flat = tblT_hbm.at[0]
        for s in range(_B_PER_W // 16):
            v = ids_v[pl.ds(16 * s, 16)]
            for d in range(D):
                widx_v[d, pl.ds(16 * s, 16)] = v + d * V
        copies = []
        for d in range(D):
            copies.append(pltpu.async_copy(
                flat.at[widx_v.at[d]],
                stage_v.at[pl.ds(d * _B_PER_W, _B_PER_W)],
                sem))
        for cp in copies:
            cp.wait()
        pltpu.sync_copy(stage_v, out_hbm.at[pl.ds(base * D, _B_PER_W * D)])

    return gather_kernel


def _dense_body(uet_ref, pft_ref, w1t_ref, b1_ref,
                w2t_ref, b2_ref, wfut_ref, wfpt_ref, bf_ref, out_ref):
    ht = jnp.maximum(
        jnp.dot(w1t_ref[...], pft_ref[...],
                preferred_element_type=jnp.float32) + b1_ref[...], 0.0)
    pet = (jnp.dot(w2t_ref[...], ht, preferred_element_type=jnp.float32)
           + b2_ref[...])
    acc = (jnp.dot(wfut_ref[...], uet_ref[...],
                   preferred_element_type=jnp.float32)
           + jnp.dot(wfpt_ref[...], pet, preferred_element_type=jnp.float32)
           + bf_ref[...])
    out_ref[...] = jnp.tanh(acc)


_BN = 2048


def _dense(uet, pft, W1t, b1c, W2t, b2c, Wfut, Wfpt, bfc):
    grid = (B // _BN,)

    def full(r, c):
        return pl.BlockSpec((r, c), lambda i: (0, 0))

    return pl.pallas_call(
        _dense_body,
        grid=grid,
        in_specs=[
            pl.BlockSpec((D, _BN), lambda i: (0, i)),
            pl.BlockSpec((P, _BN), lambda i: (0, i)),
            full(D // 2, P),
            full(D // 2, 1),
            full(D, D // 2),
            full(D, 1),
            full(D, D),
            full(D, D),
            full(D, 1),
        ],
        out_specs=pl.BlockSpec((D, _BN), lambda i: (0, i)),
        out_shape=jax.ShapeDtypeStruct((D, B), jnp.float32),
    )(uet, pft, W1t, b1c, W2t, b2c, Wfut, Wfpt, bfc)


def kernel(user_ids, profile_features, table, W1, b1, W2, b2, Wf, bf):
    ids = user_ids.astype(jnp.int32)

    out_flat = _make_sc_gather()(table.T, ids)
    uet = out_flat.reshape(_NW, D, _B_PER_W).transpose(1, 0, 2)
    uet = uet.reshape(D, B)

    pft = profile_features.T
    out_t = _dense(
        uet, pft,
        W1.T, b1.reshape(-1, 1),
        W2.T, b2.reshape(-1, 1),
        Wf[:D].T, Wf[D:].T, bf.reshape(-1, 1),
    )
    return out_t.T
